# baseline (speedup 1.0000x reference)
import functools

import jax
import jax.numpy as jnp
import numpy as np
from jax import lax
from jax.experimental import pallas as pl
from jax.experimental.pallas import tpu as pltpu, tpu_sc as plsc

B = 16
C = 255
H = 64
W = 64
A = 3
CH = 85
HW = H * W
STRIDE = 8
_ANCHORS = np.array(
    [10, 13, 16, 30, 33, 23], dtype=np.float32
).reshape(3, 2) / float(STRIDE)
_AW = tuple(float(v) for v in (_ANCHORS[:, 0] / W))
_AH = tuple(float(v) for v in (_ANCHORS[:, 1] / H))

NC = 2
NS = 16
NW = NC * NS
CHUNK = 512
NJ = HW // CHUNK
TASKS = B * A * NJ
TPW = TASKS // NW
NV = CHUNK // 16


def _sc_body(x_hbm, o_hbm, xv, ov, sem):
    wid = lax.axis_index("s") * NC + lax.axis_index("c")
    lanes = lax.iota(jnp.int32, 16)
    flanes = lanes.astype(jnp.float32)

    def task(t, carry):
        task_id = wid * TPW + t
        ba = task_id // NJ
        j = task_id % NJ
        a = ba % A
        aw = jnp.where(a == 0, _AW[0], jnp.where(a == 1, _AW[1], _AW[2]))
        ah = jnp.where(a == 0, _AH[0], jnp.where(a == 1, _AH[1], _AH[2]))

        pltpu.sync_copy(x_hbm.at[ba, :, j, :], xv)

        @plsc.parallel_loop(0, NV, 1)
        def vecs(i):
            base = (lanes + i * 16) * CH

            @plsc.parallel_loop(4, CH, 1, unroll=8)
            def chan(c):
                v = xv[c, pl.ds(i * 16, 16)]
                sig = 1.0 / (1.0 + jnp.exp(-v))
                plsc.store_scatter(ov, [base + c], sig)

        p0 = j * CHUNK

        @plsc.parallel_loop(0, NV, 1, unroll=4)
        def vec_xy(i):
            ridx = lanes + i * 16
            pos = p0 + ridx
            gx = (pos % W).astype(jnp.float32)
            gy = (pos // W).astype(jnp.float32)
            v0 = xv[0, pl.ds(i * 16, 16)]
            v1 = xv[1, pl.ds(i * 16, 16)]
            s0 = 1.0 / (1.0 + jnp.exp(-v0))
            s1 = 1.0 / (1.0 + jnp.exp(-v1))
            fidx = ridx * CH
            plsc.store_scatter(ov, [fidx], (s0 + gx) * (1.0 / W))
            plsc.store_scatter(ov, [fidx + 1], (s1 + gy) * (1.0 / H))
            v2 = xv[2, pl.ds(i * 16, 16)]
            v3 = xv[3, pl.ds(i * 16, 16)]
            plsc.store_scatter(ov, [fidx + 2], jnp.exp(v2) * aw)
            plsc.store_scatter(ov, [fidx + 3], jnp.exp(v3) * ah)


        pltpu.sync_copy(ov, o_hbm.at[ba, j, :])
        return carry

    lax.fori_loop(0, TPW, task, 0)


@functools.partial(jax.jit, static_argnames=("interpret",))
def kernel(x, interpret: bool = False):
    xr = x.reshape(B * A, CH, NJ, CHUNK)
    mesh = plsc.VectorSubcoreMesh(
        core_axis_name="c", subcore_axis_name="s",
        num_cores=NC, num_subcores=NS)
    out = pl.kernel(
        _sc_body,
        out_type=jax.ShapeDtypeStruct((B * A, NJ, CHUNK * CH), jnp.float32),
        mesh=mesh,
        scratch_types=[
            pltpu.VMEM((CH, CHUNK), jnp.float32),
            pltpu.VMEM((CHUNK * CH,), jnp.float32),
            pltpu.SemaphoreType.DMA,
        ],
        compiler_params=pltpu.CompilerParams(needs_layout_passes=False),
        interpret=interpret,
    )(xr)
    return out.reshape(B, A * HW, CH)


# speedup vs baseline: 1.0014x; 1.0014x over previous
import jax
import jax.numpy as jnp
import numpy as np
from jax import lax
from jax.experimental import pallas as pl
from jax.experimental.pallas import tpu as pltpu, tpu_sc as plsc

B = 16
C = 255
H = 64
W = 64
A = 3
CH = 85
HW = H * W
STRIDE = 8
_ANCHORS = np.array(
    [10, 13, 16, 30, 33, 23], dtype=np.float32
).reshape(3, 2) / float(STRIDE)
_AW = tuple(float(v) for v in (_ANCHORS[:, 0] / W))
_AH = tuple(float(v) for v in (_ANCHORS[:, 1] / H))

NC = 2
NS = 16
NW = NC * NS
CHUNK = 512
NJ = HW // CHUNK
TASKS = B * A * NJ
TPW = TASKS // NW
NV = CHUNK // 16


def _sc_body(x_hbm, o_hbm, xv, ov, sem):
    wid = lax.axis_index("s") * NC + lax.axis_index("c")
    lanes = lax.iota(jnp.int32, 16)

    def task(t, carry):
        task_id = wid * TPW + t
        ba = task_id // NJ
        j = task_id % NJ
        a = ba % A
        aw = jnp.where(a == 0, _AW[0], jnp.where(a == 1, _AW[1], _AW[2]))
        ah = jnp.where(a == 0, _AH[0], jnp.where(a == 1, _AH[1], _AH[2]))

        pltpu.sync_copy(x_hbm.at[ba, :, j, :], xv)

        @plsc.parallel_loop(0, NV, 1)
        def vecs(i):
            base = (lanes + i * 16) * CH

            @plsc.parallel_loop(4, CH, 1, unroll=8)
            def chan(c):
                v = xv[c, pl.ds(i * 16, 16)]
                sig = 1.0 / (1.0 + jnp.exp(-v))
                plsc.store_scatter(ov, [base + c], sig)

        p0 = j * CHUNK

        @plsc.parallel_loop(0, NV, 1, unroll=4)
        def vec_xy(i):
            ridx = lanes + i * 16
            pos = p0 + ridx
            gx = (pos % W).astype(jnp.float32)
            gy = (pos // W).astype(jnp.float32)
            v0 = xv[0, pl.ds(i * 16, 16)]
            v1 = xv[1, pl.ds(i * 16, 16)]
            s0 = 1.0 / (1.0 + jnp.exp(-v0))
            s1 = 1.0 / (1.0 + jnp.exp(-v1))
            fidx = ridx * CH
            plsc.store_scatter(ov, [fidx], (s0 + gx) * (1.0 / W))
            plsc.store_scatter(ov, [fidx + 1], (s1 + gy) * (1.0 / H))
            v2 = xv[2, pl.ds(i * 16, 16)]
            v3 = xv[3, pl.ds(i * 16, 16)]
            plsc.store_scatter(ov, [fidx + 2], jnp.exp(v2) * aw)
            plsc.store_scatter(ov, [fidx + 3], jnp.exp(v3) * ah)

        pltpu.sync_copy(ov, o_hbm.at[ba, j, :])
        return carry

    lax.fori_loop(0, TPW, task, 0)


@jax.jit
def kernel(x):
    xr = x.reshape(B * A, CH, NJ, CHUNK)
    mesh = plsc.VectorSubcoreMesh(
        core_axis_name="c", subcore_axis_name="s",
        num_cores=NC, num_subcores=NS)
    out = pl.kernel(
        _sc_body,
        out_type=jax.ShapeDtypeStruct((B * A, NJ, CHUNK * CH), jnp.float32),
        mesh=mesh,
        scratch_types=[
            pltpu.VMEM((CH, CHUNK), jnp.float32),
            pltpu.VMEM((CHUNK * CH,), jnp.float32),
            pltpu.SemaphoreType.DMA,
        ],
        compiler_params=pltpu.CompilerParams(needs_layout_passes=False),
    )(xr)
    return out.reshape(B, A * HW, CH)
